# Initial kernel scaffold; baseline (speedup 1.0000x reference)
import jax
import jax.numpy as jnp
from jax.experimental import pallas as pl


def kernel(x, cW0, cb0, cg0, cB0, cW1, cb1, cg1, cB1, cW2, cb2, cg2, cB2, cW3, cb3, cg3, cB3, fW0, fb0, fW1, fb1, fW2, fb2):
    raise NotImplementedError("write your pallas kernel here")


# speedup vs baseline: 2.9758x; 2.9758x over previous
import functools

import jax
import jax.numpy as jnp
from jax import lax
from jax.experimental import pallas as pl

B, N, K, PD = 8, 1024, 20, 3
NG = 8
EPS = 1e-5
SLOPE = 0.2


def _lrelu(x):
    return jnp.where(x >= 0, x, SLOPE * x)


def _knn_body(x_ref, idx_ref):
    xb = x_ref[0]
    xb16 = xb.astype(jnp.bfloat16)
    g = lax.dot_general(xb16, xb16, (((0,), (0,)), ((), ())),
                        preferred_element_type=jnp.float32)
    xx = jnp.sum(xb * xb, axis=0)
    inner = 2.0 * g
    s = -((xx[None, :] - inner) + xx[:, None])
    iota = lax.broadcasted_iota(jnp.int32, (N, N), 1)
    for r in range(K):
        m = jnp.max(s, axis=1, keepdims=True)
        key = jnp.where(s >= m, iota, N)
        j = jnp.min(key, axis=1)
        idx_ref[0, r, :] = j
        s = jnp.where(iota == j[:, None], -jnp.inf, s)


def _knn_topk(cur):
    c = cur.shape[1]
    return pl.pallas_call(
        _knn_body,
        grid=(B,),
        in_specs=[pl.BlockSpec((1, c, N), lambda b: (b, 0, 0))],
        out_specs=pl.BlockSpec((1, K, N), lambda b: (b, 0, 0)),
        out_shape=jax.ShapeDtypeStruct((B, K, N), jnp.int32),
    )(cur)


def _group_expand(o):
    cg = o // NG
    i0 = lax.broadcasted_iota(jnp.int32, (o, NG), 0)
    i1 = lax.broadcasted_iota(jnp.int32, (o, NG), 1)
    return (i0 // cg == i1).astype(jnp.float32)


def _hdot(a, b):
    return jnp.dot(a, b, preferred_element_type=jnp.float32,
                   precision=lax.Precision.HIGHEST)


def _edge_body(x_ref, idx_ref, w_ref, b_ref, g_ref, be_ref, out_ref, *, C, O):
    xb = x_ref[0]
    w16 = w_ref[...].astype(jnp.bfloat16)
    bcol = b_ref[...]
    idx = idx_ref[0]
    iota_j = lax.broadcasted_iota(jnp.int32, (N, N), 0)
    y_max = jnp.full((O, N), -jnp.inf, jnp.float32)
    y_min = jnp.full((O, N), jnp.inf, jnp.float32)
    s_y = jnp.zeros((O, N), jnp.float32)
    q_y = jnp.zeros((O, N), jnp.float32)
    for k in range(K):
        oh = (iota_j == idx[k, :][None, :]).astype(jnp.float32)
        xg = _hdot(xb, oh)
        fk = jnp.concatenate([xg - xb, xb], axis=0)
        yk = jnp.dot(w16, fk.astype(jnp.bfloat16),
                     preferred_element_type=jnp.float32) + bcol
        y_max = jnp.maximum(y_max, yk)
        y_min = jnp.minimum(y_min, yk)
        s_y = s_y + yk
        q_y = q_y + yk * yk
    cg = O // NG
    cnt = cg * N * K
    e = _group_expand(O)
    gsum = _hdot(e.T, jnp.sum(s_y, axis=1, keepdims=True))
    gsq = _hdot(e.T, jnp.sum(q_y, axis=1, keepdims=True))
    mean = gsum / cnt
    var = gsq / cnt - mean * mean
    sd = jnp.sqrt(var + EPS)
    mch = _hdot(e, mean)
    sdch = _hdot(e, sd)
    gam = g_ref[...]
    y = jnp.where(gam >= 0, y_max, y_min)
    out = ((y - mch) / sdch) * gam + be_ref[...]
    out_ref[0] = _lrelu(out)


def _edge(cur, idx, w, bvec, gvec, bevec):
    c = cur.shape[1]
    o = w.shape[0]
    body = functools.partial(_edge_body, C=c, O=o)
    return pl.pallas_call(
        body,
        grid=(B,),
        in_specs=[
            pl.BlockSpec((1, c, N), lambda b: (b, 0, 0)),
            pl.BlockSpec((1, K, N), lambda b: (b, 0, 0)),
            pl.BlockSpec((o, 2 * c), lambda b: (0, 0)),
            pl.BlockSpec((o, 1), lambda b: (0, 0)),
            pl.BlockSpec((o, 1), lambda b: (0, 0)),
            pl.BlockSpec((o, 1), lambda b: (0, 0)),
        ],
        out_specs=pl.BlockSpec((1, o, N), lambda b: (b, 0, 0)),
        out_shape=jax.ShapeDtypeStruct((B, o, N), jnp.float32),
    )(cur, idx, w, bvec, gvec, bevec)


def _tail_body(x1_ref, x2_ref, x3_ref, w3_ref, b3_ref, g3_ref, be3_ref,
               fw0_ref, fb0_ref, fw1_ref, fb1_ref, fw2_ref, fb2_ref, out_ref):
    w3 = w3_ref[...]
    b3 = b3_ref[...]
    cg = 256 // NG
    cnt = cg * N
    e = _group_expand(256)
    w3_16 = w3.astype(jnp.bfloat16)
    zcols = []
    for b in range(B):
        xcat = jnp.concatenate([x1_ref[b], x2_ref[b], x3_ref[b]], axis=0)
        pf = jnp.dot(w3_16, xcat.astype(jnp.bfloat16),
                     preferred_element_type=jnp.float32) + b3
        mean = _hdot(e.T, jnp.sum(pf, axis=1, keepdims=True)) / cnt
        msq = _hdot(e.T, jnp.sum(pf * pf, axis=1, keepdims=True)) / cnt
        sdg = jnp.sqrt(msq - mean * mean + EPS)
        mch = _hdot(e, mean)
        sdch = _hdot(e, sdg)
        pn = _lrelu(((pf - mch) / sdch) * g3_ref[...] + be3_ref[...])
        zmax = jnp.max(pn, axis=1, keepdims=True)
        zmean = jnp.sum(pn, axis=1, keepdims=True) / N
        zcols.append(jnp.concatenate([zmax, zmean], axis=0))
    zt = jnp.concatenate(zcols, axis=1)
    zt16 = zt.astype(jnp.bfloat16)
    h = _lrelu(jnp.dot(fw0_ref[...].astype(jnp.bfloat16), zt16,
                       preferred_element_type=jnp.float32) + fb0_ref[...])
    h = _lrelu(jnp.dot(fw1_ref[...].astype(jnp.bfloat16), h.astype(jnp.bfloat16),
                       preferred_element_type=jnp.float32) + fb1_ref[...])
    out_ref[...] = lax.dot_general(
        h.astype(jnp.bfloat16), fw2_ref[...].astype(jnp.bfloat16),
        (((0,), (1,)), ((), ())),
        preferred_element_type=jnp.float32) + fb2_ref[...]


def _tail(c1, c2, c3, w3, b3, g3, be3, fw0, fb0, fw1, fb1, fw2, fb2):
    return pl.pallas_call(
        _tail_body,
        out_shape=jax.ShapeDtypeStruct((B, 256), jnp.float32),
    )(c1, c2, c3, w3, b3[:, None], g3[:, None], be3[:, None],
      fw0, fb0[:, None], fw1, fb1[:, None], fw2, fb2[None, :])


def kernel(x, cW0, cb0, cg0, cB0, cW1, cb1, cg1, cB1, cW2, cb2, cg2, cB2,
           cW3, cb3, cg3, cB3, fW0, fb0, fW1, fb1, fW2, fb2):
    cur = jnp.swapaxes(x, 1, 2)
    res = []
    for (w, b, g, be) in [(cW0, cb0, cg0, cB0), (cW1, cb1, cg1, cB1),
                          (cW2, cb2, cg2, cB2)]:
        idx = _knn_topk(cur)
        cur = _edge(cur, idx, w, b[:, None], g[:, None], be[:, None])
        res.append(cur)
    return _tail(res[0], res[1], res[2], cW3, cb3, cg3, cB3,
                 fW0, fb0, fW1, fb1, fW2, fb2)


# speedup vs baseline: 3.7542x; 1.2616x over previous
import functools

import jax
import jax.numpy as jnp
from jax import lax
from jax.experimental import pallas as pl
from jax.experimental.pallas import tpu as pltpu
from jax.experimental.pallas import tpu_sc as plsc

B, N, K, PD = 8, 1024, 20, 3
NG = 8
EPS = 1e-5
SLOPE = 0.2
M = B * N * K
CH = 128
NW = 32
CPW = M // (NW * CH)


def _lrelu(x):
    return jnp.where(x >= 0, x, SLOPE * x)


def _hdot(a, b):
    return jnp.dot(a, b, preferred_element_type=jnp.float32,
                   precision=lax.Precision.HIGHEST)


def _ddot(a, b):
    return jnp.dot(a.astype(jnp.bfloat16), b.astype(jnp.bfloat16),
                   preferred_element_type=jnp.float32)


def _group_expand(o):
    cg = o // NG
    i0 = lax.broadcasted_iota(jnp.int32, (o, NG), 0)
    i1 = lax.broadcasted_iota(jnp.int32, (o, NG), 1)
    return (i0 // cg == i1).astype(jnp.float32)


def _knn_body(x_ref, idx_ref):
    xb = x_ref[0]
    xb16 = xb.astype(jnp.bfloat16)
    g = lax.dot_general(xb16, xb16, (((0,), (0,)), ((), ())),
                        preferred_element_type=jnp.float32)
    xx = jnp.sum(xb * xb, axis=0)
    inner = 2.0 * g
    s = -((xx[None, :] - inner) + xx[:, None])
    iota = lax.broadcasted_iota(jnp.int32, (N, N), 1)
    base = pl.program_id(0) * N
    for r in range(K):
        m = jnp.max(s, axis=1, keepdims=True)
        key = jnp.where(s >= m, iota, N)
        j = jnp.min(key, axis=1)
        idx_ref[0, r, :] = j + base
        s = jnp.where(iota == j[:, None], -jnp.inf, s)


def _knn_topk(cur_t):
    c = cur_t.shape[1]
    return pl.pallas_call(
        _knn_body,
        grid=(B,),
        in_specs=[pl.BlockSpec((1, c, N), lambda b: (b, 0, 0))],
        out_specs=pl.BlockSpec((1, K, N), lambda b: (b, 0, 0)),
        out_shape=jax.ShapeDtypeStruct((B, K, N), jnp.int32),
    )(cur_t)


def _make_sc_gather(c):
    mesh = plsc.VectorSubcoreMesh(core_axis_name="c", subcore_axis_name="s")

    @functools.partial(
        pl.kernel, mesh=mesh,
        compiler_params=pltpu.CompilerParams(use_tc_tiling_on_sc=False),
        out_type=(jax.ShapeDtypeStruct((M, c), jnp.float32),
                  jax.ShapeDtypeStruct((M, c), jnp.float32)),
        scratch_types=[
            pltpu.VMEM((CPW, CH), jnp.int32),
            pltpu.VMEM((CPW, CH), jnp.int32),
            pltpu.VMEM((CH, c), jnp.float32),
            pltpu.VMEM((CH, c), jnp.float32),
            pltpu.SemaphoreType.DMA,
            pltpu.SemaphoreType.DMA,
        ],
    )
    def sc_gather(tbl_hbm, idxn_hbm, idxs_hbm, outn_hbm, outs_hbm,
                  idxn_v, idxs_v, bufn, bufs, semn, sems):
        wid = lax.axis_index("s") * 2 + lax.axis_index("c")
        crow = wid * CPW
        pltpu.sync_copy(idxn_hbm.at[pl.ds(crow, CPW)], idxn_v)
        pltpu.sync_copy(idxs_hbm.at[pl.ds(crow, CPW)], idxs_v)

        def body(ch, carry):
            cn = pltpu.async_copy(tbl_hbm.at[idxn_v.at[ch]], bufn, semn)
            cs = pltpu.async_copy(tbl_hbm.at[idxs_v.at[ch]], bufs, sems)
            cn.wait()
            cs.wait()
            base = (crow + ch) * CH
            pltpu.sync_copy(bufn, outn_hbm.at[pl.ds(base, CH)])
            pltpu.sync_copy(bufs, outs_hbm.at[pl.ds(base, CH)])
            return carry

        lax.fori_loop(0, CPW, body, 0)

    return sc_gather


def _sc_gather(tbl, idxn, idxs, c):
    return _make_sc_gather(c)(tbl, idxn, idxs)


def _edge_body(xg_ref, xs_ref, wt_ref, b_ref, g_ref, be_ref, out_ref, *, C, O):
    xg = xg_ref[...]
    xs = xs_ref[...]
    f = jnp.concatenate([xg - xs, xs], axis=1)
    y = _ddot(f, wt_ref[...]) + b_ref[...]
    s_y = jnp.sum(y, axis=0, keepdims=True)
    q_y = jnp.sum(y * y, axis=0, keepdims=True)
    cg = O // NG
    cnt = cg * N * K
    e = _group_expand(O)
    mean = _hdot(s_y, e) / cnt
    var = _hdot(q_y, e) / cnt - mean * mean
    sd = jnp.sqrt(var + EPS)
    mch = _hdot(mean, e.T)
    sdch = _hdot(sd, e.T)
    y3 = y.reshape(N, K, O)
    y_max = jnp.max(y3, axis=1)
    y_min = jnp.min(y3, axis=1)
    gam = g_ref[...]
    y_sel = jnp.where(gam >= 0, y_max, y_min)
    out_ref[0] = _lrelu(((y_sel - mch) / sdch) * gam + be_ref[...])


def _edge(xg, xs, wt, bvec, gvec, bevec, o):
    c = xg.shape[1]
    body = functools.partial(_edge_body, C=c, O=o)
    return pl.pallas_call(
        body,
        grid=(B,),
        in_specs=[
            pl.BlockSpec((N * K, c), lambda b: (b, 0)),
            pl.BlockSpec((N * K, c), lambda b: (b, 0)),
            pl.BlockSpec((2 * c, o), lambda b: (0, 0)),
            pl.BlockSpec((1, o), lambda b: (0, 0)),
            pl.BlockSpec((1, o), lambda b: (0, 0)),
            pl.BlockSpec((1, o), lambda b: (0, 0)),
        ],
        out_specs=pl.BlockSpec((1, N, o), lambda b: (b, 0, 0)),
        out_shape=jax.ShapeDtypeStruct((B, N, o), jnp.float32),
    )(xg, xs, wt, bvec, gvec, bevec)


def _tail_body(x1_ref, x2_ref, x3_ref, w3t_ref, b3_ref, g3_ref, be3_ref,
               fw0_ref, fb0_ref, fw1_ref, fb1_ref, fw2_ref, fb2_ref, out_ref):
    w3t = w3t_ref[...]
    b3 = b3_ref[...]
    cg = 256 // NG
    cnt = cg * N
    e = _group_expand(256)
    zrows = []
    for b in range(B):
        xcat = jnp.concatenate([x1_ref[b], x2_ref[b], x3_ref[b]], axis=1)
        pf = _ddot(xcat, w3t) + b3
        mean = _hdot(jnp.sum(pf, axis=0, keepdims=True), e) / cnt
        msq = _hdot(jnp.sum(pf * pf, axis=0, keepdims=True), e) / cnt
        sdg = jnp.sqrt(msq - mean * mean + EPS)
        mch = _hdot(mean, e.T)
        sdch = _hdot(sdg, e.T)
        pn = _lrelu(((pf - mch) / sdch) * g3_ref[...] + be3_ref[...])
        zmax = jnp.max(pn, axis=0, keepdims=True)
        zmean = jnp.sum(pn, axis=0, keepdims=True) / N
        zrows.append(jnp.concatenate([zmax, zmean], axis=1))
    z = jnp.concatenate(zrows, axis=0)
    h = _lrelu(lax.dot_general(
        z.astype(jnp.bfloat16), fw0_ref[...].astype(jnp.bfloat16),
        (((1,), (1,)), ((), ())),
        preferred_element_type=jnp.float32) + fb0_ref[...])
    h = _lrelu(lax.dot_general(
        h.astype(jnp.bfloat16), fw1_ref[...].astype(jnp.bfloat16),
        (((1,), (1,)), ((), ())),
        preferred_element_type=jnp.float32) + fb1_ref[...])
    out_ref[...] = lax.dot_general(
        h.astype(jnp.bfloat16), fw2_ref[...].astype(jnp.bfloat16),
        (((1,), (1,)), ((), ())),
        preferred_element_type=jnp.float32) + fb2_ref[...]


def _tail(c1, c2, c3, w3, b3, g3, be3, fw0, fb0, fw1, fb1, fw2, fb2):
    return pl.pallas_call(
        _tail_body,
        out_shape=jax.ShapeDtypeStruct((B, 256), jnp.float32),
    )(c1, c2, c3, w3.T, b3[None, :], g3[None, :], be3[None, :],
      fw0, fb0[None, :], fw1, fb1[None, :], fw2, fb2[None, :])


def _pad_w(w, c, cpad):
    o = w.shape[0]
    wp = jnp.zeros((o, 2 * cpad), w.dtype)
    wp = wp.at[:, :c].set(w[:, :c])
    wp = wp.at[:, cpad:cpad + c].set(w[:, c:])
    return wp


def kernel(x, cW0, cb0, cg0, cB0, cW1, cb1, cg1, cB1, cW2, cb2, cg2, cB2,
           cW3, cb3, cg3, cB3, fW0, fb0, fW1, fb1, fW2, fb2):
    idxs = (jnp.arange(M, dtype=jnp.int32) // K).reshape(-1, CH)
    cur = x
    res = []
    for li, (w, b, g, be) in enumerate([(cW0, cb0, cg0, cB0),
                                        (cW1, cb1, cg1, cB1),
                                        (cW2, cb2, cg2, cB2)]):
        c = cur.shape[2]
        o = w.shape[0]
        idxn = jnp.swapaxes(_knn_topk(jnp.swapaxes(cur, 1, 2)),
                            1, 2).reshape(-1, CH)
        if c == PD:
            cpad = 16
            tbl = jnp.pad(cur, ((0, 0), (0, 0), (0, cpad - c)))
            wt = _pad_w(w, c, cpad).T
        else:
            cpad, tbl, wt = c, cur, w.T
        xg, xs = _sc_gather(tbl.reshape(B * N, cpad), idxn, idxs, cpad)
        cur = _edge(xg, xs, wt, b[None, :], g[None, :], be[None, :], o)
        res.append(cur)
    return _tail(res[0], res[1], res[2], cW3, cb3, cg3, cB3,
                 fW0, fb0, fW1, fb1, fW2, fb2)


# speedup vs baseline: 5.5906x; 1.4892x over previous
import functools

import jax
import jax.numpy as jnp
from jax import lax
from jax.experimental import pallas as pl
from jax.experimental.pallas import tpu as pltpu
from jax.experimental.pallas import tpu_sc as plsc

B, N, K, PD = 8, 1024, 20, 3
NG = 8
EPS = 1e-5
SLOPE = 0.2
M = B * N * K
CH = 128
NW = 32
CPW = M // (NW * CH)


def _lrelu(x):
    return jnp.where(x >= 0, x, SLOPE * x)


def _hdot(a, b):
    return jnp.dot(a, b, preferred_element_type=jnp.float32,
                   precision=lax.Precision.HIGHEST)


def _ddot(a, b):
    return jnp.dot(a.astype(jnp.bfloat16), b.astype(jnp.bfloat16),
                   preferred_element_type=jnp.float32)


def _group_expand(o):
    cg = o // NG
    i0 = lax.broadcasted_iota(jnp.int32, (o, NG), 0)
    i1 = lax.broadcasted_iota(jnp.int32, (o, NG), 1)
    return (i0 // cg == i1).astype(jnp.float32)


def _knn_body(x_ref, idx_ref):
    xb = x_ref[0]
    xb16 = xb.astype(jnp.bfloat16)
    g = lax.dot_general(xb16, xb16, (((0,), (0,)), ((), ())),
                        preferred_element_type=jnp.float32)
    xx = jnp.sum(xb * xb, axis=0)
    inner = 2.0 * g
    s = -((xx[None, :] - inner) + xx[:, None])
    iota = lax.broadcasted_iota(jnp.int32, (N, N), 1)
    base = pl.program_id(0) * N
    for r in range(K):
        m = jnp.max(s, axis=1, keepdims=True)
        key = jnp.where(s >= m, iota, N)
        j = jnp.min(key, axis=1)
        idx_ref[0, r, :] = j + base
        s = jnp.where(iota == j[:, None], -jnp.inf, s)


def _knn_topk(cur_t):
    c = cur_t.shape[1]
    return pl.pallas_call(
        _knn_body,
        grid=(B,),
        in_specs=[pl.BlockSpec((1, c, N), lambda b: (b, 0, 0))],
        out_specs=pl.BlockSpec((1, K, N), lambda b: (b, 0, 0)),
        out_shape=jax.ShapeDtypeStruct((B, K, N), jnp.int32),
    )(cur_t)


def _make_sc_gather(c):
    mesh = plsc.VectorSubcoreMesh(core_axis_name="c", subcore_axis_name="s")

    @functools.partial(
        pl.kernel, mesh=mesh,
        compiler_params=pltpu.CompilerParams(use_tc_tiling_on_sc=False),
        out_type=jax.ShapeDtypeStruct((M, c), jnp.float32),
        scratch_types=[
            pltpu.VMEM((CPW, CH), jnp.int32),
            pltpu.VMEM((CH, c), jnp.float32),
            pltpu.VMEM((CH, c), jnp.float32),
            pltpu.SemaphoreType.DMA,
            pltpu.SemaphoreType.DMA,
        ],
    )
    def sc_gather(tbl_hbm, idx_hbm, out_hbm, idx_v, buf0, buf1, sem0, sem1):
        wid = lax.axis_index("s") * 2 + lax.axis_index("c")
        crow = wid * CPW
        pltpu.sync_copy(idx_hbm.at[pl.ds(crow, CPW)], idx_v)
        pltpu.async_copy(tbl_hbm.at[idx_v.at[0]], buf0, sem0)

        def body(i, carry):
            g = 2 * i
            pltpu.async_copy(tbl_hbm.at[idx_v.at[g + 1]], buf1, sem1)
            pltpu.make_async_copy(tbl_hbm.at[idx_v.at[g]], buf0, sem0).wait()
            pltpu.sync_copy(buf0, out_hbm.at[pl.ds((crow + g) * CH, CH)])

            @pl.when(g + 2 < CPW)
            def _():
                pltpu.async_copy(tbl_hbm.at[idx_v.at[g + 2]], buf0, sem0)

            pltpu.make_async_copy(tbl_hbm.at[idx_v.at[g + 1]], buf1,
                                  sem1).wait()
            pltpu.sync_copy(buf1, out_hbm.at[pl.ds((crow + g + 1) * CH, CH)])
            return carry

        lax.fori_loop(0, CPW // 2, body, 0)

    return sc_gather


def _sc_gather(tbl, idxn, c):
    return _make_sc_gather(c)(tbl, idxn)


def _edge_body(x_ref, xg_ref, wt_ref, b_ref, g_ref, be_ref, out_ref, *, C, O):
    xt = x_ref[0]
    wt = wt_ref[...]
    bcol = b_ref[...]
    y_max = jnp.full((N, O), -jnp.inf, jnp.float32)
    y_min = jnp.full((N, O), jnp.inf, jnp.float32)
    s_y = jnp.zeros((1, O), jnp.float32)
    q_y = jnp.zeros((1, O), jnp.float32)
    for k in range(K):
        xk = xg_ref[0, pl.ds(k * N, N), :]
        f = jnp.concatenate([xk - xt, xt], axis=1)
        y = _ddot(f, wt) + bcol
        y_max = jnp.maximum(y_max, y)
        y_min = jnp.minimum(y_min, y)
        s_y = s_y + jnp.sum(y, axis=0, keepdims=True)
        q_y = q_y + jnp.sum(y * y, axis=0, keepdims=True)
    cg = O // NG
    cnt = cg * N * K
    e = _group_expand(O)
    mean = _hdot(s_y, e) / cnt
    var = _hdot(q_y, e) / cnt - mean * mean
    sd = jnp.sqrt(var + EPS)
    mch = _hdot(mean, e.T)
    sdch = _hdot(sd, e.T)
    gam = g_ref[...]
    y_sel = jnp.where(gam >= 0, y_max, y_min)
    out_ref[0] = _lrelu(((y_sel - mch) / sdch) * gam + be_ref[...])


def _edge(cur, xg, wt, bvec, gvec, bevec, o):
    c = xg.shape[2]
    body = functools.partial(_edge_body, C=c, O=o)
    return pl.pallas_call(
        body,
        grid=(B,),
        in_specs=[
            pl.BlockSpec((1, N, c), lambda b: (b, 0, 0)),
            pl.BlockSpec((1, K * N, c), lambda b: (b, 0, 0)),
            pl.BlockSpec((2 * c, o), lambda b: (0, 0)),
            pl.BlockSpec((1, o), lambda b: (0, 0)),
            pl.BlockSpec((1, o), lambda b: (0, 0)),
            pl.BlockSpec((1, o), lambda b: (0, 0)),
        ],
        out_specs=pl.BlockSpec((1, N, o), lambda b: (b, 0, 0)),
        out_shape=jax.ShapeDtypeStruct((B, N, o), jnp.float32),
    )(cur, xg, wt, bvec, gvec, bevec)


def _tail_body(x1_ref, x2_ref, x3_ref, w3t_ref, b3_ref, g3_ref, be3_ref,
               fw0_ref, fb0_ref, fw1_ref, fb1_ref, fw2_ref, fb2_ref, out_ref):
    w3t = w3t_ref[...]
    b3 = b3_ref[...]
    cg = 256 // NG
    cnt = cg * N
    e = _group_expand(256)
    zrows = []
    for b in range(B):
        xcat = jnp.concatenate([x1_ref[b], x2_ref[b], x3_ref[b]], axis=1)
        pf = _ddot(xcat, w3t) + b3
        mean = _hdot(jnp.sum(pf, axis=0, keepdims=True), e) / cnt
        msq = _hdot(jnp.sum(pf * pf, axis=0, keepdims=True), e) / cnt
        sdg = jnp.sqrt(msq - mean * mean + EPS)
        mch = _hdot(mean, e.T)
        sdch = _hdot(sdg, e.T)
        pn = _lrelu(((pf - mch) / sdch) * g3_ref[...] + be3_ref[...])
        zmax = jnp.max(pn, axis=0, keepdims=True)
        zmean = jnp.sum(pn, axis=0, keepdims=True) / N
        zrows.append(jnp.concatenate([zmax, zmean], axis=1))
    z = jnp.concatenate(zrows, axis=0)
    h = _lrelu(lax.dot_general(
        z.astype(jnp.bfloat16), fw0_ref[...].astype(jnp.bfloat16),
        (((1,), (1,)), ((), ())),
        preferred_element_type=jnp.float32) + fb0_ref[...])
    h = _lrelu(lax.dot_general(
        h.astype(jnp.bfloat16), fw1_ref[...].astype(jnp.bfloat16),
        (((1,), (1,)), ((), ())),
        preferred_element_type=jnp.float32) + fb1_ref[...])
    out_ref[...] = lax.dot_general(
        h.astype(jnp.bfloat16), fw2_ref[...].astype(jnp.bfloat16),
        (((1,), (1,)), ((), ())),
        preferred_element_type=jnp.float32) + fb2_ref[...]


def _tail(c1, c2, c3, w3, b3, g3, be3, fw0, fb0, fw1, fb1, fw2, fb2):
    return pl.pallas_call(
        _tail_body,
        out_shape=jax.ShapeDtypeStruct((B, 256), jnp.float32),
    )(c1, c2, c3, w3.T, b3[None, :], g3[None, :], be3[None, :],
      fw0, fb0[None, :], fw1, fb1[None, :], fw2, fb2[None, :])


def _pad_w(w, c, cpad):
    o = w.shape[0]
    wp = jnp.zeros((o, 2 * cpad), w.dtype)
    wp = wp.at[:, :c].set(w[:, :c])
    wp = wp.at[:, cpad:cpad + c].set(w[:, c:])
    return wp


def kernel(x, cW0, cb0, cg0, cB0, cW1, cb1, cg1, cB1, cW2, cb2, cg2, cB2,
           cW3, cb3, cg3, cB3, fW0, fb0, fW1, fb1, fW2, fb2):
    cur = x
    res = []
    for li, (w, b, g, be) in enumerate([(cW0, cb0, cg0, cB0),
                                        (cW1, cb1, cg1, cB1),
                                        (cW2, cb2, cg2, cB2)]):
        c = cur.shape[2]
        o = w.shape[0]
        idxn = _knn_topk(jnp.swapaxes(cur, 1, 2)).reshape(-1, CH)
        if c == PD:
            cpad = 16
            tbl = jnp.pad(cur, ((0, 0), (0, 0), (0, cpad - c)))
            wt = _pad_w(w, c, cpad).T
        else:
            cpad, tbl, wt = c, cur, w.T
        xg = _sc_gather(tbl.reshape(B * N, cpad), idxn, cpad)
        cur = _edge(tbl, xg.reshape(B, K * N, cpad), wt,
                    b[None, :], g[None, :], be[None, :], o)
        res.append(cur)
    return _tail(res[0], res[1], res[2], cW3, cb3, cg3, cB3,
                 fW0, fb0, fW1, fb1, fW2, fb2)
